# Initial kernel scaffold; baseline (speedup 1.0000x reference)
import jax
import jax.numpy as jnp
from jax.experimental import pallas as pl


def kernel(X):
    raise NotImplementedError("write your pallas kernel here")


# speedup vs baseline: 10.1994x; 10.1994x over previous
import jax
import jax.numpy as jnp
from jax import lax
from jax.experimental import pallas as pl
from jax.experimental.pallas import tpu as pltpu
from jax.experimental.pallas import tpu_sc as plsc

L = 16
ROWS = 128
D = 32768
CHUNKS = D // L
NC, NS = 2, 16
NW = NC * NS
RPW = ROWS // NW
UNROLL = 8
N_BISECT = 40


def _take(v, idx):
    return v.at[idx].get(mode="promise_in_bounds")


def _all_max(v, iota):
    for s in (8, 4, 2, 1):
        v = jnp.maximum(v, _take(v, iota ^ s))
    return v


def _all_sum(v, iota):
    for s in (8, 4, 2, 1):
        v = v + _take(v, iota ^ s)
    return v


def _prefix_sum(v, iota):
    for s in (1, 2, 4, 8):
        shifted = _take(v, jnp.maximum(iota - s, 0))
        v = v + jnp.where(iota >= s, shifted, 0)
    return v


def _tsallis_body(x_hbm, y_hbm, row_v, surv_v):
    cid = lax.axis_index("c")
    sid = lax.axis_index("s")
    wid = sid * NC + cid
    iota = lax.iota(jnp.int32, L)
    fifteen = jnp.full((L,), 15, jnp.int32)

    def per_row(j, carry):
        row = wid * RPW + j
        pltpu.sync_copy(x_hbm.at[row], row_v)

        def amax_body(it, acc):
            for u in range(UNROLL):
                i = it * UNROLL + u
                acc = jnp.maximum(acc, row_v[pl.ds(i * L, L)])
            return acc

        macc = lax.fori_loop(0, CHUNKS // UNROLL, amax_body,
                             jnp.full((L,), -jnp.inf, jnp.float32))
        m = _all_max(macc, iota)
        thresh = m - 2.0

        def compact_body(it, off):
            for u in range(UNROLL):
                i = it * UNROLL + u
                x = row_v[pl.ds(i * L, L)]
                mask = x > thresh
                xs = (x - m) * 0.5
                ones = jnp.where(mask, jnp.full((L,), 1, jnp.int32),
                                 jnp.full((L,), 0, jnp.int32))
                prefix = _prefix_sum(ones, iota)
                idx = jnp.where(mask, off + prefix - 1,
                                jnp.full((L,), D + L - 1, jnp.int32))
                plsc.store_scatter(surv_v, [idx], xs)
                off = off + _take(prefix, fifteen)
            return off

        off = lax.fori_loop(0, CHUNKS // UNROLL, compact_body,
                            jnp.zeros((L,), jnp.int32))
        nsurv = off[0]
        surv_v[pl.ds(nsurv, L)] = jnp.full((L,), -4.0, jnp.float32)
        nchunks = (nsurv + L - 1) // L

        def bis_body(it, lohi):
            lo, hi = lohi
            mid = 0.5 * (lo + hi)

            def facc(i, acc):
                r = jnp.maximum(surv_v[pl.ds(i * L, L)] - mid, 0.0)
                return acc + r * r

            fv = lax.fori_loop(0, nchunks, facc, jnp.zeros((L,), jnp.float32))
            ge = _all_sum(fv, iota) >= 1.0
            return (jnp.where(ge, mid, lo), jnp.where(ge, hi, mid))

        lo, hi = lax.fori_loop(
            0, N_BISECT, bis_body,
            (jnp.full((L,), -1.0, jnp.float32), jnp.zeros((L,), jnp.float32)))
        tau = 0.5 * (lo + hi)
        c = m + 2.0 * tau

        def out_body(it, carry2):
            for u in range(UNROLL):
                i = it * UNROLL + u
                r = jnp.maximum((row_v[pl.ds(i * L, L)] - c) * 0.5, 0.0)
                row_v[pl.ds(i * L, L)] = r * r
            return carry2

        lax.fori_loop(0, CHUNKS // UNROLL, out_body, 0)
        pltpu.sync_copy(row_v, y_hbm.at[row])
        return carry

    lax.fori_loop(0, RPW, per_row, 0)


@jax.jit
def kernel(X):
    k = pl.kernel(
        _tsallis_body,
        out_type=jax.ShapeDtypeStruct((ROWS, D), jnp.float32),
        mesh=plsc.VectorSubcoreMesh(core_axis_name="c", subcore_axis_name="s"),
        compiler_params=pltpu.CompilerParams(needs_layout_passes=False),
        scratch_types=[
            pltpu.VMEM((D,), jnp.float32),
            pltpu.VMEM((D + L,), jnp.float32),
        ],
    )
    return k(X)


# speedup vs baseline: 25.6447x; 2.5143x over previous
import jax
import jax.numpy as jnp
from jax import lax
from jax.experimental import pallas as pl
from jax.experimental.pallas import tpu as pltpu
from jax.experimental.pallas import tpu_sc as plsc

L = 16
ROWS = 128
D = 32768
CHUNKS = D // L
NC, NS = 2, 16
NW = NC * NS
RPW = ROWS // NW
N_BISECT = 40


def _take(v, idx):
    return v.at[idx].get(mode="promise_in_bounds")


def _all_max(v, iota):
    for s in (8, 4, 2, 1):
        v = jnp.maximum(v, _take(v, iota ^ s))
    return v


def _all_sum(v, iota):
    for s in (8, 4, 2, 1):
        v = v + _take(v, iota ^ s)
    return v


def _prefix_sum(v, iota):
    for s in (1, 2, 4, 8):
        shifted = _take(v, jnp.maximum(iota - s, 0))
        v = v + jnp.where(iota >= s, shifted, 0)
    return v


def _tsallis_body(x_hbm, y_hbm, row_v, surv_v):
    cid = lax.axis_index("c")
    sid = lax.axis_index("s")
    wid = sid * NC + cid
    iota = lax.iota(jnp.int32, L)
    fifteen = jnp.full((L,), 15, jnp.int32)

    def per_row(j, carry):
        row = wid * RPW + j
        pltpu.sync_copy(x_hbm.at[row], row_v)

        @plsc.parallel_loop(0, CHUNKS, unroll=8,
                            carry=jnp.full((L,), -jnp.inf, jnp.float32))
        def macc(i, acc):
            return jnp.maximum(acc, row_v[pl.ds(i * L, L)])

        m = _all_max(macc, iota)
        thresh = m - 2.0

        @plsc.parallel_loop(0, CHUNKS, unroll=8,
                            carry=jnp.zeros((L,), jnp.int32))
        def off(i, off_c):
            x = row_v[pl.ds(i * L, L)]
            mask = x > thresh
            xs = (x - m) * 0.5
            ones = jnp.where(mask, jnp.full((L,), 1, jnp.int32),
                             jnp.full((L,), 0, jnp.int32))
            prefix = _prefix_sum(ones, iota)
            idx = jnp.where(mask, off_c + prefix - 1,
                            jnp.full((L,), D + L - 1, jnp.int32))
            plsc.store_scatter(surv_v, [idx], xs)
            return off_c + _take(prefix, fifteen)

        nsurv = off[0]
        surv_v[pl.ds(nsurv, L)] = jnp.full((L,), -4.0, jnp.float32)
        nchunks = (nsurv + L - 1) // L

        def bis_body(it, lohi):
            lo, hi = lohi
            mid = 0.5 * (lo + hi)

            @plsc.parallel_loop(0, nchunks, unroll=4,
                                carry=jnp.zeros((L,), jnp.float32))
            def fv(i, acc):
                r = jnp.maximum(surv_v[pl.ds(i * L, L)] - mid, 0.0)
                return acc + r * r

            ge = _all_sum(fv, iota) >= 1.0
            return (jnp.where(ge, mid, lo), jnp.where(ge, hi, mid))

        lo, hi = lax.fori_loop(
            0, N_BISECT, bis_body,
            (jnp.full((L,), -1.0, jnp.float32), jnp.zeros((L,), jnp.float32)))
        tau = 0.5 * (lo + hi)
        c = m + 2.0 * tau

        @plsc.parallel_loop(0, CHUNKS, unroll=8)
        def _(i):
            r = jnp.maximum((row_v[pl.ds(i * L, L)] - c) * 0.5, 0.0)
            row_v[pl.ds(i * L, L)] = r * r

        pltpu.sync_copy(row_v, y_hbm.at[row])
        return carry

    lax.fori_loop(0, RPW, per_row, 0)


@jax.jit
def kernel(X):
    k = pl.kernel(
        _tsallis_body,
        out_type=jax.ShapeDtypeStruct((ROWS, D), jnp.float32),
        mesh=plsc.VectorSubcoreMesh(core_axis_name="c", subcore_axis_name="s"),
        compiler_params=pltpu.CompilerParams(needs_layout_passes=False),
        scratch_types=[
            pltpu.VMEM((D,), jnp.float32),
            pltpu.VMEM((D + L,), jnp.float32),
        ],
    )
    return k(X)


# speedup vs baseline: 26.8880x; 1.0485x over previous
import jax
import jax.numpy as jnp
from jax import lax
from jax.experimental import pallas as pl
from jax.experimental.pallas import tpu as pltpu
from jax.experimental.pallas import tpu_sc as plsc

L = 16
ROWS = 128
D = 32768
CHUNKS = D // L
NC, NS = 2, 16
NW = NC * NS
RPW = ROWS // NW
N_BISECT = 40


def _take(v, idx):
    return v.at[idx].get(mode="promise_in_bounds")


def _all_max(v, iota):
    for s in (8, 4, 2, 1):
        v = jnp.maximum(v, _take(v, iota ^ s))
    return v


def _all_sum(v, iota):
    for s in (8, 4, 2, 1):
        v = v + _take(v, iota ^ s)
    return v


def _prefix_sum(v, iota):
    for s in (1, 2, 4, 8):
        shifted = _take(v, jnp.maximum(iota - s, 0))
        v = v + jnp.where(iota >= s, shifted, 0)
    return v


def _tsallis_body(x_hbm, y_hbm, row0_v, row1_v, surv_v, sin0, sin1, sout0,
                  sout1):
    cid = lax.axis_index("c")
    sid = lax.axis_index("s")
    wid = sid * NC + cid
    iota = lax.iota(jnp.int32, L)
    fifteen = jnp.full((L,), 15, jnp.int32)
    sin = (sin0, sin1)
    sout = (sout0, sout1)
    bufs = (row0_v, row1_v)

    def compute(row_v):
        @plsc.parallel_loop(0, CHUNKS, unroll=8,
                            carry=jnp.full((L,), -jnp.inf, jnp.float32))
        def macc(i, acc):
            return jnp.maximum(acc, row_v[pl.ds(i * L, L)])

        m = _all_max(macc, iota)
        thresh = m - 2.0

        @plsc.parallel_loop(0, CHUNKS, unroll=8,
                            carry=jnp.zeros((L,), jnp.int32))
        def off(i, off_c):
            x = row_v[pl.ds(i * L, L)]
            mask = x > thresh
            xs = (x - m) * 0.5
            ones = jnp.where(mask, jnp.full((L,), 1, jnp.int32),
                             jnp.full((L,), 0, jnp.int32))
            prefix = _prefix_sum(ones, iota)
            idx = jnp.where(mask, off_c + prefix - 1,
                            jnp.full((L,), D + L - 1, jnp.int32))
            plsc.store_scatter(surv_v, [idx], xs)
            return off_c + _take(prefix, fifteen)

        nsurv = off[0]
        surv_v[pl.ds(nsurv, L)] = jnp.full((L,), -4.0, jnp.float32)
        nchunks = (nsurv + L - 1) // L

        def bis_body(it, lohi):
            lo, hi = lohi
            mid = 0.5 * (lo + hi)

            @plsc.parallel_loop(0, nchunks, unroll=4,
                                carry=jnp.zeros((L,), jnp.float32))
            def fv(i, acc):
                r = jnp.maximum(surv_v[pl.ds(i * L, L)] - mid, 0.0)
                return acc + r * r

            ge = _all_sum(fv, iota) >= 1.0
            return (jnp.where(ge, mid, lo), jnp.where(ge, hi, mid))

        lo, hi = lax.fori_loop(
            0, N_BISECT, bis_body,
            (jnp.full((L,), -1.0, jnp.float32), jnp.zeros((L,), jnp.float32)))
        tau = 0.5 * (lo + hi)
        c = m + 2.0 * tau

        @plsc.parallel_loop(0, CHUNKS, unroll=8)
        def _(i):
            r = jnp.maximum((row_v[pl.ds(i * L, L)] - c) * 0.5, 0.0)
            row_v[pl.ds(i * L, L)] = r * r

    base = wid * RPW
    copies_out = [None] * RPW
    copy_in = [None] * RPW
    copy_in[0] = pltpu.async_copy(x_hbm.at[base], bufs[0], sin[0])
    for j in range(RPW):
        b = j % 2
        if j + 1 < RPW:
            if j - 1 >= 0:
                copies_out[j - 1].wait()
            copy_in[j + 1] = pltpu.async_copy(
                x_hbm.at[base + j + 1], bufs[(j + 1) % 2], sin[(j + 1) % 2])
        copy_in[j].wait()
        compute(bufs[b])
        copies_out[j] = pltpu.async_copy(bufs[b], y_hbm.at[base + j], sout[b])
    copies_out[RPW - 2].wait()
    copies_out[RPW - 1].wait()


@jax.jit
def kernel(X):
    k = pl.kernel(
        _tsallis_body,
        out_type=jax.ShapeDtypeStruct((ROWS, D), jnp.float32),
        mesh=plsc.VectorSubcoreMesh(core_axis_name="c", subcore_axis_name="s"),
        compiler_params=pltpu.CompilerParams(needs_layout_passes=False),
        scratch_types=[
            pltpu.VMEM((D,), jnp.float32),
            pltpu.VMEM((D,), jnp.float32),
            pltpu.VMEM((D + L,), jnp.float32),
            pltpu.SemaphoreType.DMA,
            pltpu.SemaphoreType.DMA,
            pltpu.SemaphoreType.DMA,
            pltpu.SemaphoreType.DMA,
        ],
    )
    return k(X)


# speedup vs baseline: 34.7840x; 1.2937x over previous
import jax
import jax.numpy as jnp
from jax import lax
from jax.experimental import pallas as pl
from jax.experimental.pallas import tpu as pltpu
from jax.experimental.pallas import tpu_sc as plsc

L = 16
ROWS = 128
D = 32768
CHUNKS = D // L
NC, NS = 2, 16
NW = NC * NS
RPW = ROWS // NW
N_BISECT = 40


def _take(v, idx):
    return v.at[idx].get(mode="promise_in_bounds")


def _all_max(v, iota):
    for s in (8, 4, 2, 1):
        v = jnp.maximum(v, _take(v, iota ^ s))
    return v


def _all_sum(v, iota):
    for s in (8, 4, 2, 1):
        v = v + _take(v, iota ^ s)
    return v


def _prefix_sum(v, iota):
    for s in (1, 2, 4, 8):
        shifted = _take(v, jnp.maximum(iota - s, 0))
        v = v + jnp.where(iota >= s, shifted, 0)
    return v


def _tsallis_body(x_hbm, y_hbm, row0_v, row1_v, surv_v, sin0, sin1, sout0,
                  sout1):
    cid = lax.axis_index("c")
    sid = lax.axis_index("s")
    wid = sid * NC + cid
    iota = lax.iota(jnp.int32, L)
    fifteen = jnp.full((L,), 15, jnp.int32)
    sin = (sin0, sin1)
    sout = (sout0, sout1)
    bufs = (row0_v, row1_v)

    def compute(row_v):
        @plsc.parallel_loop(0, CHUNKS, unroll=8,
                            carry=jnp.full((L,), -jnp.inf, jnp.float32))
        def macc(i, acc):
            return jnp.maximum(acc, row_v[pl.ds(i * L, L)])

        m = _all_max(macc, iota)
        thresh = m - 2.0

        @plsc.parallel_loop(0, CHUNKS, unroll=8,
                            carry=jnp.zeros((L,), jnp.int32))
        def off(i, off_c):
            x = row_v[pl.ds(i * L, L)]
            mask = x > thresh
            xs = (x - m) * 0.5
            sk, _sv, _om = plsc.sort_key_val(xs, xs, mask=mask,
                                             descending=True)
            cnt = plsc.all_reduce_population_count(mask)
            idx = jnp.where(iota < cnt, off_c + iota,
                            jnp.full((L,), D + L - 1, jnp.int32))
            plsc.store_scatter(surv_v, [idx], sk)
            return off_c + cnt

        nsurv = off[0]
        surv_v[pl.ds(nsurv, L)] = jnp.full((L,), -4.0, jnp.float32)
        nchunks = (nsurv + L - 1) // L

        def bis_body(it, lohi):
            lo, hi = lohi
            mid = 0.5 * (lo + hi)

            @plsc.parallel_loop(0, nchunks, unroll=4,
                                carry=jnp.zeros((L,), jnp.float32))
            def fv(i, acc):
                r = jnp.maximum(surv_v[pl.ds(i * L, L)] - mid, 0.0)
                return acc + r * r

            ge = _all_sum(fv, iota) >= 1.0
            return (jnp.where(ge, mid, lo), jnp.where(ge, hi, mid))

        lo, hi = lax.fori_loop(
            0, N_BISECT, bis_body,
            (jnp.full((L,), -1.0, jnp.float32), jnp.zeros((L,), jnp.float32)))
        tau = 0.5 * (lo + hi)
        c = m + 2.0 * tau

        @plsc.parallel_loop(0, CHUNKS, unroll=8)
        def _(i):
            r = jnp.maximum((row_v[pl.ds(i * L, L)] - c) * 0.5, 0.0)
            row_v[pl.ds(i * L, L)] = r * r

    base = wid * RPW
    copies_out = [None] * RPW
    copy_in = [None] * RPW
    copy_in[0] = pltpu.async_copy(x_hbm.at[base], bufs[0], sin[0])
    for j in range(RPW):
        b = j % 2
        if j + 1 < RPW:
            if j - 1 >= 0:
                copies_out[j - 1].wait()
            copy_in[j + 1] = pltpu.async_copy(
                x_hbm.at[base + j + 1], bufs[(j + 1) % 2], sin[(j + 1) % 2])
        copy_in[j].wait()
        compute(bufs[b])
        copies_out[j] = pltpu.async_copy(bufs[b], y_hbm.at[base + j], sout[b])
    copies_out[RPW - 2].wait()
    copies_out[RPW - 1].wait()


@jax.jit
def kernel(X):
    k = pl.kernel(
        _tsallis_body,
        out_type=jax.ShapeDtypeStruct((ROWS, D), jnp.float32),
        mesh=plsc.VectorSubcoreMesh(core_axis_name="c", subcore_axis_name="s"),
        compiler_params=pltpu.CompilerParams(needs_layout_passes=False),
        scratch_types=[
            pltpu.VMEM((D,), jnp.float32),
            pltpu.VMEM((D,), jnp.float32),
            pltpu.VMEM((D + L,), jnp.float32),
            pltpu.SemaphoreType.DMA,
            pltpu.SemaphoreType.DMA,
            pltpu.SemaphoreType.DMA,
            pltpu.SemaphoreType.DMA,
        ],
    )
    return k(X)


# speedup vs baseline: 35.5836x; 1.0230x over previous
import jax
import jax.numpy as jnp
from jax import lax
from jax.experimental import pallas as pl
from jax.experimental.pallas import tpu as pltpu
from jax.experimental.pallas import tpu_sc as plsc

L = 16
ROWS = 128
D = 32768
CHUNKS = D // L
NC, NS = 2, 16
NW = NC * NS
RPW = ROWS // NW
N_BISECT = 32


def _take(v, idx):
    return v.at[idx].get(mode="promise_in_bounds")


def _all_max(v, iota):
    for s in (8, 4, 2, 1):
        v = jnp.maximum(v, _take(v, iota ^ s))
    return v


def _all_sum(v, iota):
    for s in (8, 4, 2, 1):
        v = v + _take(v, iota ^ s)
    return v


def _prefix_sum(v, iota):
    for s in (1, 2, 4, 8):
        shifted = _take(v, jnp.maximum(iota - s, 0))
        v = v + jnp.where(iota >= s, shifted, 0)
    return v


def _tsallis_body(x_hbm, y_hbm, row0_v, row1_v, surv_v, sin0, sin1, sout0,
                  sout1):
    cid = lax.axis_index("c")
    sid = lax.axis_index("s")
    wid = sid * NC + cid
    iota = lax.iota(jnp.int32, L)
    fifteen = jnp.full((L,), 15, jnp.int32)
    sin = (sin0, sin1)
    sout = (sout0, sout1)
    bufs = (row0_v, row1_v)

    def compute(row_v):
        @plsc.parallel_loop(0, CHUNKS, unroll=8,
                            carry=jnp.full((L,), -jnp.inf, jnp.float32))
        def macc(i, acc):
            return jnp.maximum(acc, row_v[pl.ds(i * L, L)])

        m = _all_max(macc, iota)
        thresh = m - 2.0

        @plsc.parallel_loop(0, CHUNKS, unroll=8,
                            carry=jnp.zeros((L,), jnp.int32))
        def off(i, off_c):
            x = row_v[pl.ds(i * L, L)]
            mask = x > thresh
            xs = (x - m) * 0.5
            sk, _sv, _om = plsc.sort_key_val(xs, xs, mask=mask,
                                             descending=True)
            cnt = plsc.all_reduce_population_count(mask)
            idx = jnp.where(iota < cnt, off_c + iota,
                            jnp.full((L,), D + L - 1, jnp.int32))
            plsc.store_scatter(surv_v, [idx], sk)
            return off_c + cnt

        nsurv = off[0]
        surv_v[pl.ds(nsurv, L)] = jnp.full((L,), -4.0, jnp.float32)
        nchunks = (nsurv + L - 1) // L

        def bis_body(it, lohi):
            lo, hi = lohi
            mid = 0.5 * (lo + hi)

            @plsc.parallel_loop(0, nchunks, unroll=8,
                                carry=jnp.zeros((L,), jnp.float32))
            def fv(i, acc):
                r = jnp.maximum(surv_v[pl.ds(i * L, L)] - mid, 0.0)
                return acc + r * r

            ge = _all_sum(fv, iota) >= 1.0
            return (jnp.where(ge, mid, lo), jnp.where(ge, hi, mid))

        lo, hi = lax.fori_loop(
            0, N_BISECT, bis_body,
            (jnp.full((L,), -1.0, jnp.float32), jnp.zeros((L,), jnp.float32)))
        tau = 0.5 * (lo + hi)
        c = m + 2.0 * tau

        @plsc.parallel_loop(0, CHUNKS, unroll=8)
        def _(i):
            r = jnp.maximum((row_v[pl.ds(i * L, L)] - c) * 0.5, 0.0)
            row_v[pl.ds(i * L, L)] = r * r

    base = wid * RPW
    copies_out = [None] * RPW
    copy_in = [None] * RPW
    copy_in[0] = pltpu.async_copy(x_hbm.at[base], bufs[0], sin[0])
    for j in range(RPW):
        b = j % 2
        if j + 1 < RPW:
            if j - 1 >= 0:
                copies_out[j - 1].wait()
            copy_in[j + 1] = pltpu.async_copy(
                x_hbm.at[base + j + 1], bufs[(j + 1) % 2], sin[(j + 1) % 2])
        copy_in[j].wait()
        compute(bufs[b])
        copies_out[j] = pltpu.async_copy(bufs[b], y_hbm.at[base + j], sout[b])
    copies_out[RPW - 2].wait()
    copies_out[RPW - 1].wait()


@jax.jit
def kernel(X):
    k = pl.kernel(
        _tsallis_body,
        out_type=jax.ShapeDtypeStruct((ROWS, D), jnp.float32),
        mesh=plsc.VectorSubcoreMesh(core_axis_name="c", subcore_axis_name="s"),
        compiler_params=pltpu.CompilerParams(needs_layout_passes=False),
        scratch_types=[
            pltpu.VMEM((D,), jnp.float32),
            pltpu.VMEM((D,), jnp.float32),
            pltpu.VMEM((D + L,), jnp.float32),
            pltpu.SemaphoreType.DMA,
            pltpu.SemaphoreType.DMA,
            pltpu.SemaphoreType.DMA,
            pltpu.SemaphoreType.DMA,
        ],
    )
    return k(X)
